# Initial kernel scaffold; baseline (speedup 1.0000x reference)
import jax
import jax.numpy as jnp
from jax.experimental import pallas as pl


def kernel(input_x, emb_table, pos_table):
    raise NotImplementedError("write your pallas kernel here")


# speedup vs baseline: 3.5981x; 3.5981x over previous
import functools

import jax
import jax.numpy as jnp
from jax import lax
from jax.experimental import pallas as pl
from jax.experimental.pallas import tpu as pltpu
from jax.experimental.pallas import tpu_sc as plsc

NC = 2
NS = 16
NW = NC * NS

CHUNK = 100
HID = 64
LANES = 16


def _body(idx_hbm, pos_hbm, emb_hbm, out_hbm, idx_v, pos_v, buf_v, gsem):
    n_chunks_total = idx_hbm.shape[0]
    per_w = n_chunks_total // NW
    wid = lax.axis_index("s") * NC + lax.axis_index("c")
    base = wid * per_w

    pltpu.sync_copy(idx_hbm.at[pl.ds(base, per_w)], idx_v)
    pltpu.sync_copy(pos_hbm, pos_v)

    def chunk_step(c, carry):
        pltpu.async_copy(emb_hbm.at[idx_v.at[c]], buf_v, gsem).wait()

        def row_add(r, carry2):
            for k in range(HID // LANES):
                plsc.addupdate(buf_v.at[r, pl.ds(k * LANES, LANES)],
                               pos_v[r, pl.ds(k * LANES, LANES)])
            return carry2

        lax.fori_loop(0, CHUNK, row_add, 0, unroll=4)

        pltpu.sync_copy(buf_v, out_hbm.at[base + c])
        return carry

    lax.fori_loop(0, per_w, chunk_step, 0)


def kernel(input_x, emb_table, pos_table):
    batch, seq_len = input_x.shape
    total = batch * seq_len
    n_chunks = total // CHUNK
    idx = input_x.reshape(n_chunks, CHUNK).astype(jnp.int32)
    pos_rep = jnp.concatenate([pos_table[:seq_len], pos_table[:seq_len]], axis=0)

    mesh = plsc.VectorSubcoreMesh(
        core_axis_name="c", subcore_axis_name="s", num_cores=NC, num_subcores=NS
    )
    out = pl.kernel(
        _body,
        out_type=jax.ShapeDtypeStruct((n_chunks, CHUNK, HID), jnp.float32),
        mesh=mesh,
        scratch_types=[
            pltpu.VMEM((n_chunks // NW, CHUNK), jnp.int32),
            pltpu.VMEM((CHUNK, HID), jnp.float32),
            pltpu.VMEM((CHUNK, HID), jnp.float32),
            pltpu.SemaphoreType.DMA,
        ],
        compiler_params=pltpu.CompilerParams(use_tc_tiling_on_sc=False),
    )(idx, pos_rep, emb_table)
    return out.reshape(batch, seq_len, HID)


# speedup vs baseline: 4.5645x; 1.2686x over previous
import functools

import jax
import jax.numpy as jnp
from jax import lax
from jax.experimental import pallas as pl
from jax.experimental.pallas import tpu as pltpu
from jax.experimental.pallas import tpu_sc as plsc

NC = 2
NS = 16
NW = NC * NS

CHUNK = 100
HID = 64
LANES = 16
NBUF = 4
LOOK = 2


def _body(idx_hbm, pos_hbm, emb_hbm, out_hbm, idx_v, pos_v, buf_v, gsem, ssem):
    n_chunks_total = idx_hbm.shape[0]
    per_w = n_chunks_total // NW
    wid = lax.axis_index("s") * NC + lax.axis_index("c")
    base = wid * per_w

    pltpu.sync_copy(idx_hbm.at[pl.ds(base, per_w)], idx_v)
    pltpu.sync_copy(pos_hbm, pos_v)

    def start_gather(c, slot):
        pltpu.async_copy(emb_hbm.at[idx_v.at[c]], buf_v.at[slot], gsem.at[slot])

    def wait_gather(c, slot):
        pltpu.make_async_copy(
            emb_hbm.at[idx_v.at[c]], buf_v.at[slot], gsem.at[slot]
        ).wait()

    def start_store(c, slot):
        pltpu.async_copy(buf_v.at[slot], out_hbm.at[base + c], ssem.at[slot])

    def wait_store(c, slot):
        pltpu.make_async_copy(
            buf_v.at[slot], out_hbm.at[base + c], ssem.at[slot]
        ).wait()

    for b in range(LOOK):
        start_gather(b, b)

    def outer(g, carry):
        c0 = g * NBUF
        for b in range(NBUF):
            c = c0 + b
            slot = b
            nslot = (b + LOOK) % NBUF

            @pl.when(c + LOOK - NBUF >= 0)
            def _():
                wait_store(c + LOOK - NBUF, nslot)

            @pl.when(c + LOOK < per_w)
            def _():
                start_gather(c + LOOK, nslot)

            wait_gather(c, slot)

            def row_add(r, carry2):
                for k in range(HID // LANES):
                    plsc.addupdate(buf_v.at[slot, r, pl.ds(k * LANES, LANES)],
                                   pos_v[r, pl.ds(k * LANES, LANES)])
                return carry2

            lax.fori_loop(0, CHUNK, row_add, 0, unroll=4)

            start_store(c, slot)
        return carry

    lax.fori_loop(0, per_w // NBUF, outer, 0)

    for i in range(LOOK):
        c = per_w - LOOK + i
        wait_store(c, c % NBUF)


def kernel(input_x, emb_table, pos_table):
    batch, seq_len = input_x.shape
    total = batch * seq_len
    n_chunks = total // CHUNK
    idx = input_x.reshape(n_chunks, CHUNK).astype(jnp.int32)
    pos_rep = jnp.concatenate([pos_table[:seq_len], pos_table[:seq_len]], axis=0)

    mesh = plsc.VectorSubcoreMesh(
        core_axis_name="c", subcore_axis_name="s", num_cores=NC, num_subcores=NS
    )
    out = pl.kernel(
        _body,
        out_type=jax.ShapeDtypeStruct((n_chunks, CHUNK, HID), jnp.float32),
        mesh=mesh,
        scratch_types=[
            pltpu.VMEM((n_chunks // NW, CHUNK), jnp.int32),
            pltpu.VMEM((CHUNK, HID), jnp.float32),
            pltpu.VMEM((NBUF, CHUNK, HID), jnp.float32),
            pltpu.SemaphoreType.DMA((NBUF,)),
            pltpu.SemaphoreType.DMA((NBUF,)),
        ],
        compiler_params=pltpu.CompilerParams(use_tc_tiling_on_sc=False),
    )(idx, pos_rep, emb_table)
    return out.reshape(batch, seq_len, HID)
